# baseline (speedup 1.0000x reference)
import jax
import jax.numpy as jnp
from jax.experimental import pallas as pl
from jax.experimental.pallas import tpu as pltpu


def _prep_body(trow_ref, ts_ref, es_ref, w_ref, a_ref, d_ref):
    m = ts_ref.shape[1]
    B = trow_ref.shape[1]
    ts = ts_ref[:, :]
    lo = ts[0, 0]
    hi = ts[0, m - 1]

    tr = trow_ref[:, :]
    trc = jnp.clip(tr, lo, hi)
    idxc = jnp.zeros(tr.shape, jnp.int32)
    for mm in range(m):
        idxc += (ts[0, mm] < trc).astype(jnp.int32)
    idxc = jnp.clip(idxc, 1, m - 1)
    t0 = jnp.zeros(tr.shape, jnp.float32)
    t1 = jnp.zeros(tr.shape, jnp.float32)
    for mm in range(m):
        t0 = jnp.where(idxc - 1 == mm, ts[0, mm], t0)
        t1 = jnp.where(idxc == mm, ts[0, mm], t1)
    w_ref[:, :] = (trc - t0) / (t1 - t0 + 1e-12)

    rows = jax.lax.broadcasted_iota(jnp.int32, (m, B), 0)
    p0 = (rows == (idxc - 1)).astype(jnp.float32)
    p1 = (rows == idxc).astype(jnp.float32)
    es = es_ref[:, :]
    dn = (((0,), (0,)), ((), ()))
    e0 = jax.lax.dot_general(p0, es, dn,
                             preferred_element_type=jnp.float32)
    e1 = jax.lax.dot_general(p1, es, dn,
                             preferred_element_type=jnp.float32)
    a_ref[:, :] = e0
    d_ref[:, :] = e1 - e0


_NBUF = 4
_CH = 1024


def _stream_body(a_ref, d_ref, w_ref, o_hbm, buf, sems):
    nsteps = pl.num_programs(0)
    step = pl.program_id(0)
    slot = jax.lax.rem(step, _NBUF)
    B = w_ref.shape[1]

    @pl.when(step >= _NBUF)
    def _():
        pltpu.make_async_copy(buf.at[slot], o_hbm.at[pl.ds(0, _CH)],
                              sems.at[slot]).wait()

    a = a_ref[pl.ds(step * _CH, _CH), :]
    d = d_ref[pl.ds(step * _CH, _CH), :]
    w = w_ref[:, :]
    buf[slot, :, :] = a + d * w

    pltpu.make_async_copy(buf.at[slot], o_hbm.at[pl.ds(step * _CH, _CH)],
                          sems.at[slot]).start()

    @pl.when(step == nsteps - 1)
    def _():
        for q in range(_NBUF):
            pltpu.make_async_copy(buf.at[q], o_hbm.at[pl.ds(0, _CH)],
                                  sems.at[q]).wait()


def kernel(t, ts, Es):
    B = t.shape[0]
    m = ts.shape[0]
    k = Es.shape[1]
    R = B * k

    ts2 = ts.reshape(1, m)
    trow = t.reshape(1, B)

    w, A, D = pl.pallas_call(
        _prep_body,
        out_shape=(
            jax.ShapeDtypeStruct((1, B), jnp.float32),
            jax.ShapeDtypeStruct((B, k), jnp.float32),
            jax.ShapeDtypeStruct((B, k), jnp.float32),
        ),
    )(trow, ts2, Es)

    a = A.reshape(R, 1)
    d = D.reshape(R, 1)

    q = pl.pallas_call(
        _stream_body,
        grid=(R // _CH,),
        in_specs=[
            pl.BlockSpec(memory_space=pltpu.MemorySpace.VMEM),
            pl.BlockSpec(memory_space=pltpu.MemorySpace.VMEM),
            pl.BlockSpec(memory_space=pltpu.MemorySpace.VMEM),
        ],
        out_specs=pl.BlockSpec(memory_space=pl.ANY),
        out_shape=jax.ShapeDtypeStruct((R, B), jnp.float32),
        scratch_shapes=[
            pltpu.VMEM((_NBUF, _CH, B), jnp.float32),
            pltpu.SemaphoreType.DMA((_NBUF,)),
        ],
    )(a, d, w)

    return q.reshape(B, k, B).transpose(0, 2, 1)
